# Initial kernel scaffold; baseline (speedup 1.0000x reference)
import jax
import jax.numpy as jnp
from jax.experimental import pallas as pl


def kernel(x, gate_W, gate_b, expert_W, expert_b):
    raise NotImplementedError("write your pallas kernel here")


# speedup vs baseline: 1.1073x; 1.1073x over previous
import functools

import jax
import jax.numpy as jnp
from jax.experimental import pallas as pl
from jax.experimental.pallas import tpu as pltpu

TOP_K = 2


def _moe_dense_body(x_ref, gw_ref, gb_ref, ew_ref, eb_ref,
                    out_ref, logits_ref, wpe_ref, *, expert_num):
    e = pl.program_id(1)

    @pl.when(e == 0)
    def _router():
        x = x_ref[...]
        logits = jax.lax.dot_general(
            x, gw_ref[...], (((1,), (1,)), ((), ())),
            preferred_element_type=jnp.float32) + gb_ref[...]
        logits_ref[...] = logits
        m1 = jnp.max(logits, axis=-1, keepdims=True)
        a1 = jnp.argmax(logits, axis=-1)
        cols = jax.lax.broadcasted_iota(jnp.int32, logits.shape, 1)
        mask1 = cols == a1[:, None]
        neg = jnp.full_like(logits, -jnp.inf)
        logits2 = jnp.where(mask1, neg, logits)
        m2 = jnp.max(logits2, axis=-1, keepdims=True)
        a2 = jnp.argmax(logits2, axis=-1)
        mask2 = cols == a2[:, None]
        w1 = 1.0 / (1.0 + jnp.exp(m2 - m1))
        w2 = 1.0 - w1
        wpe_ref[...] = jnp.where(mask1, w1, 0.0) + jnp.where(mask2, w2, 0.0)

    x = x_ref[...]
    cur = jax.lax.dot_general(
        x, ew_ref[0], (((1,), (1,)), ((), ())),
        preferred_element_type=jnp.float32) + eb_ref[0]
    wpe = wpe_ref[...]
    ecols = jax.lax.broadcasted_iota(jnp.int32, wpe.shape, 1)
    w_e = jnp.sum(jnp.where(ecols == e, wpe, 0.0), axis=1, keepdims=True)
    contrib = cur * w_e

    @pl.when(e == 0)
    def _init():
        out_ref[...] = contrib

    @pl.when(e > 0)
    def _acc():
        out_ref[...] += contrib


def kernel(x, gate_W, gate_b, expert_W, expert_b):
    batch, seq, hidden = x.shape
    expert_num = gate_W.shape[0]
    T = batch * seq
    xs = x.reshape(T, hidden)
    TM = 512
    grid = (T // TM, expert_num)

    out, logits = pl.pallas_call(
        functools.partial(_moe_dense_body, expert_num=expert_num),
        grid=grid,
        in_specs=[
            pl.BlockSpec((TM, hidden), lambda i, e: (i, 0)),
            pl.BlockSpec((expert_num, hidden), lambda i, e: (0, 0)),
            pl.BlockSpec((1, expert_num), lambda i, e: (0, 0)),
            pl.BlockSpec((1, hidden, hidden), lambda i, e: (e, 0, 0)),
            pl.BlockSpec((1, 1, hidden), lambda i, e: (e, 0, 0)),
        ],
        out_specs=[
            pl.BlockSpec((TM, hidden), lambda i, e: (i, 0)),
            pl.BlockSpec((TM, expert_num), lambda i, e: (i, 0)),
        ],
        out_shape=[
            jax.ShapeDtypeStruct((T, hidden), jnp.float32),
            jax.ShapeDtypeStruct((T, expert_num), jnp.float32),
        ],
        scratch_shapes=[pltpu.VMEM((TM, expert_num), jnp.float32)],
    )(xs, gate_W, gate_b.reshape(1, expert_num), expert_W,
      expert_b.reshape(expert_num, 1, hidden))

    return out.reshape(batch, seq, hidden), logits


# speedup vs baseline: 1.1375x; 1.0272x over previous
import functools

import jax
import jax.numpy as jnp
from jax.experimental import pallas as pl
from jax.experimental.pallas import tpu as pltpu

TOP_K = 2


def _moe_dense_body(x_ref, gw_ref, gb_ref, ew_ref, eb_ref,
                    out_ref, logits_ref, wpe_ref, *, expert_num):
    e = pl.program_id(1)

    @pl.when(e == 0)
    def _router():
        x = x_ref[...]
        logits = jax.lax.dot_general(
            x, gw_ref[...], (((1,), (1,)), ((), ())),
            preferred_element_type=jnp.float32) + gb_ref[...]
        logits_ref[...] = logits
        m1 = jnp.max(logits, axis=-1, keepdims=True)
        a1 = jnp.argmax(logits, axis=-1)
        cols = jax.lax.broadcasted_iota(jnp.int32, logits.shape, 1)
        mask1 = cols == a1[:, None]
        neg = jnp.full_like(logits, -jnp.inf)
        logits2 = jnp.where(mask1, neg, logits)
        m2 = jnp.max(logits2, axis=-1, keepdims=True)
        a2 = jnp.argmax(logits2, axis=-1)
        mask2 = cols == a2[:, None]
        w1 = 1.0 / (1.0 + jnp.exp(m2 - m1))
        w2 = 1.0 - w1
        wpe_ref[...] = jnp.where(mask1, w1, 0.0) + jnp.where(mask2, w2, 0.0)

    x = x_ref[...]
    cur = jax.lax.dot_general(
        x.astype(jnp.bfloat16), ew_ref[0], (((1,), (1,)), ((), ())),
        preferred_element_type=jnp.float32) + eb_ref[0]
    wpe = wpe_ref[...]
    ecols = jax.lax.broadcasted_iota(jnp.int32, wpe.shape, 1)
    w_e = jnp.sum(jnp.where(ecols == e, wpe, 0.0), axis=1, keepdims=True)
    contrib = cur * w_e

    @pl.when(e == 0)
    def _init():
        out_ref[...] = contrib

    @pl.when(e > 0)
    def _acc():
        out_ref[...] += contrib


def kernel(x, gate_W, gate_b, expert_W, expert_b):
    batch, seq, hidden = x.shape
    expert_num = gate_W.shape[0]
    T = batch * seq
    xs = x.reshape(T, hidden)
    TM = 512
    grid = (T // TM, expert_num)

    out, logits = pl.pallas_call(
        functools.partial(_moe_dense_body, expert_num=expert_num),
        grid=grid,
        in_specs=[
            pl.BlockSpec((TM, hidden), lambda i, e: (i, 0)),
            pl.BlockSpec((expert_num, hidden), lambda i, e: (0, 0)),
            pl.BlockSpec((1, expert_num), lambda i, e: (0, 0)),
            pl.BlockSpec((1, hidden, hidden), lambda i, e: (e, 0, 0)),
            pl.BlockSpec((1, 1, hidden), lambda i, e: (e, 0, 0)),
        ],
        out_specs=[
            pl.BlockSpec((TM, hidden), lambda i, e: (i, 0)),
            pl.BlockSpec((TM, expert_num), lambda i, e: (i, 0)),
        ],
        out_shape=[
            jax.ShapeDtypeStruct((T, hidden), jnp.float32),
            jax.ShapeDtypeStruct((T, expert_num), jnp.float32),
        ],
        scratch_shapes=[pltpu.VMEM((TM, expert_num), jnp.float32)],
    )(xs, gate_W, gate_b.reshape(1, expert_num),
      expert_W.astype(jnp.bfloat16),
      expert_b.reshape(expert_num, 1, hidden))

    return out.reshape(batch, seq, hidden), logits
